# baseline (speedup 1.0000x reference)
import functools

import numpy as np
import jax
import jax.numpy as jnp
from jax import lax
from jax.experimental import pallas as pl
from jax.experimental.pallas import tpu as pltpu
from jax.experimental.pallas import tpu_sc as plsc

_B = 1024
_LPOS = 20
_LSESS = 50
_NCLASS = 100000
_NEG_CAND = 256

_NC = 2
_NS = 16
_L = 16
_NW = _NC * _NS

_POS_PAD = 32
_N_NEG = _B * _B
_N_POS = _B * _POS_PAD
_N_TOT = _N_NEG + _N_POS
_CHUNK = 128
_N_CHUNKS = _N_TOT // _CHUNK
_CHUNKS_PER_W = _N_CHUNKS // _NW


def _cand_table():
    nkey = jax.random.key(42)
    nkeys = jax.random.split(nkey, _B)
    cand = jax.vmap(
        lambda k: jax.random.randint(k, (_NEG_CAND,), 0, _NCLASS))(nkeys)
    return cand.astype(jnp.int32)


def _stage1_body(labels_ref, sessions_ref, cand_ref, negidx_ref, posidx_ref):
    labels = labels_ref[...]
    sessions = sessions_ref[...]
    cand = cand_ref[...]

    bad = jnp.zeros((_B, _NEG_CAND), dtype=jnp.bool_)
    for j in range(_LPOS):
        bad = bad | (cand == labels[:, j][:, None])
    for j in range(_LSESS):
        bad = bad | (cand == sessions[:, j][:, None])

    kiota = lax.broadcasted_iota(jnp.int32, (_B, _NEG_CAND), 1)
    km = jnp.where(bad, _NEG_CAND, kiota)
    first = jnp.min(km, axis=1)
    first = jnp.where(first == _NEG_CAND, 0, first)
    onehot = kiota == first[:, None]
    neg = jnp.sum(jnp.where(onehot, cand, 0), axis=1)

    riota = lax.broadcasted_iota(jnp.int32, (_B, _B), 0)
    negidx_ref[...] = riota * _NCLASS + neg[None, :]

    labels_pad = jnp.concatenate(
        [labels, jnp.zeros((_B, _POS_PAD - _LPOS), jnp.int32)], axis=1)
    riota2 = lax.broadcasted_iota(jnp.int32, (_B, _POS_PAD), 0)
    posidx_ref[...] = riota2 * _NCLASS + labels_pad


def _stage1(labels, sessions, cand):
    return pl.pallas_call(
        _stage1_body,
        out_shape=(
            jax.ShapeDtypeStruct((_B, _B), jnp.int32),
            jax.ShapeDtypeStruct((_B, _POS_PAD), jnp.int32),
        ),
    )(labels, sessions, cand)


def _stage2_body(table_hbm, idx_hbm, out_hbm, fidx_v, vals_v, sem):
    wid = lax.axis_index("s") * _NC + lax.axis_index("c")

    def body(c, carry):
        base = (wid * _CHUNKS_PER_W + c) * _CHUNK
        pltpu.sync_copy(idx_hbm.at[pl.ds(base, _CHUNK)], fidx_v)
        pltpu.async_copy(table_hbm.at[fidx_v], vals_v, sem).wait()
        pltpu.sync_copy(vals_v, out_hbm.at[pl.ds(base, _CHUNK)])
        return carry

    lax.fori_loop(0, _CHUNKS_PER_W, body, 0)


def _stage2(table, idx_all):
    mesh = plsc.VectorSubcoreMesh(core_axis_name="c", subcore_axis_name="s")
    k = functools.partial(
        pl.kernel,
        mesh=mesh,
        out_type=jax.ShapeDtypeStruct((_N_TOT,), jnp.float32),
        scratch_types=[
            pltpu.VMEM((_CHUNK,), jnp.int32),
            pltpu.VMEM((_CHUNK,), jnp.float32),
            pltpu.SemaphoreType.DMA,
        ],
    )(_stage2_body)
    return k(table, idx_all)


def _stage3_body(scores_ref, out_ref):
    s = scores_ref[...]
    sneg = s[: _N_NEG // 128, :]
    spos = s[_N_NEG // 128:, :]
    neg_prob = jax.nn.sigmoid(sneg)
    neg_e = -jnp.log(1.0 - neg_prob + 1e-10)
    neg_sum = jnp.sum(neg_e)
    col = lax.broadcasted_iota(jnp.int32, (_N_POS // 128, 128), 1)
    valid = (col % _POS_PAD) < _LPOS
    pos_prob = jax.nn.sigmoid(spos)
    pos_e = -jnp.log(pos_prob + 1e-10)
    pos_sum = jnp.sum(jnp.where(valid, pos_e, 0.0))
    loss = neg_sum / _B + pos_sum / (_B * _LPOS)
    out_ref[...] = jnp.reshape(loss, (1, 1))


def _stage3(scores):
    return pl.pallas_call(
        _stage3_body,
        out_shape=jax.ShapeDtypeStruct((1, 1), jnp.float32),
    )(scores)


def kernel(outputs, labels, sessions):
    cand = _cand_table()
    negidx, posidx = _stage1(labels, sessions, cand)
    idx_all = jnp.concatenate([negidx.reshape(-1), posidx.reshape(-1)])
    table = outputs.reshape(_B * _NCLASS)
    scores = _stage2(table, idx_all)
    loss = _stage3(scores.reshape(_N_TOT // 128, 128))
    return loss.reshape(())


# speedup vs baseline: 1.2932x; 1.2932x over previous
import functools

import numpy as np
import jax
import jax.numpy as jnp
from jax import lax
from jax.experimental import pallas as pl
from jax.experimental.pallas import tpu as pltpu
from jax.experimental.pallas import tpu_sc as plsc

_B = 1024
_LPOS = 20
_LSESS = 50
_NCLASS = 100000
_NEG_CAND = 256

_NC = 2
_NS = 16
_L = 16
_NW = _NC * _NS

_POS_PAD = 32
_N_NEG = _B * _B
_N_POS = _B * _POS_PAD
_N_TOT = _N_NEG + _N_POS
_CHUNK = 128
_N_CHUNKS = _N_TOT // _CHUNK
_CHUNKS_PER_W = _N_CHUNKS // _NW


def _cand_table():
    nkey = jax.random.key(42)
    nkeys = jax.random.split(nkey, _B)
    cand = jax.vmap(
        lambda k: jax.random.randint(k, (_NEG_CAND,), 0, _NCLASS))(nkeys)
    return cand.astype(jnp.int32)


def _stage1_body(labels_ref, sessions_ref, cand_ref, negidx_ref, posidx_ref):
    labels = labels_ref[...]
    sessions = sessions_ref[...]
    cand = cand_ref[...]

    bad = jnp.zeros((_B, _NEG_CAND), dtype=jnp.bool_)
    for j in range(_LPOS):
        bad = bad | (cand == labels[:, j][:, None])
    for j in range(_LSESS):
        bad = bad | (cand == sessions[:, j][:, None])

    kiota = lax.broadcasted_iota(jnp.int32, (_B, _NEG_CAND), 1)
    km = jnp.where(bad, _NEG_CAND, kiota)
    first = jnp.min(km, axis=1)
    first = jnp.where(first == _NEG_CAND, 0, first)
    onehot = kiota == first[:, None]
    neg = jnp.sum(jnp.where(onehot, cand, 0), axis=1)

    riota = lax.broadcasted_iota(jnp.int32, (_B, _B), 0)
    negidx_ref[...] = riota * _NCLASS + neg[None, :]

    labels_pad = jnp.concatenate(
        [labels, jnp.zeros((_B, _POS_PAD - _LPOS), jnp.int32)], axis=1)
    riota2 = lax.broadcasted_iota(jnp.int32, (_B, _POS_PAD), 0)
    posidx_ref[...] = riota2 * _NCLASS + labels_pad


def _stage1(labels, sessions, cand):
    return pl.pallas_call(
        _stage1_body,
        out_shape=(
            jax.ShapeDtypeStruct((_B, _B), jnp.int32),
            jax.ShapeDtypeStruct((_B, _POS_PAD), jnp.int32),
        ),
    )(labels, sessions, cand)


_UNROLL = 8
_N_OUTER = _CHUNKS_PER_W // _UNROLL


def _stage2_body(table_hbm, idx_hbm, out_hbm, idx_v, vals_v, sem):
    wid = lax.axis_index("s") * _NC + lax.axis_index("c")
    row0 = wid * _CHUNKS_PER_W

    pltpu.sync_copy(idx_hbm.at[pl.ds(row0, _CHUNKS_PER_W)], idx_v)

    def fire(c, carry):
        for j in range(_UNROLL):
            r = c * _UNROLL + j
            pltpu.async_copy(table_hbm.at[idx_v.at[r]], vals_v.at[r], sem)
        return carry

    lax.fori_loop(0, _N_OUTER, fire, 0)

    pltpu.make_async_copy(
        out_hbm.at[pl.ds(row0, _CHUNKS_PER_W)], vals_v, sem).wait()

    pltpu.sync_copy(vals_v, out_hbm.at[pl.ds(row0, _CHUNKS_PER_W)])


def _stage2(table, idx_all):
    mesh = plsc.VectorSubcoreMesh(core_axis_name="c", subcore_axis_name="s")
    k = functools.partial(
        pl.kernel,
        mesh=mesh,
        out_type=jax.ShapeDtypeStruct((_N_CHUNKS, _CHUNK), jnp.float32),
        scratch_types=[
            pltpu.VMEM((_CHUNKS_PER_W, _CHUNK), jnp.int32),
            pltpu.VMEM((_CHUNKS_PER_W, _CHUNK), jnp.float32),
            pltpu.SemaphoreType.DMA,
        ],
    )(_stage2_body)
    return k(table, idx_all)


def _stage3_body(scores_ref, out_ref):
    s = scores_ref[...]
    sneg = s[: _N_NEG // 128, :]
    spos = s[_N_NEG // 128:, :]
    neg_prob = jax.nn.sigmoid(sneg)
    neg_e = -jnp.log(1.0 - neg_prob + 1e-10)
    neg_sum = jnp.sum(neg_e)
    col = lax.broadcasted_iota(jnp.int32, (_N_POS // 128, 128), 1)
    valid = (col % _POS_PAD) < _LPOS
    pos_prob = jax.nn.sigmoid(spos)
    pos_e = -jnp.log(pos_prob + 1e-10)
    pos_sum = jnp.sum(jnp.where(valid, pos_e, 0.0))
    loss = neg_sum / _B + pos_sum / (_B * _LPOS)
    out_ref[...] = jnp.reshape(loss, (1, 1))


def _stage3(scores):
    return pl.pallas_call(
        _stage3_body,
        out_shape=jax.ShapeDtypeStruct((1, 1), jnp.float32),
    )(scores)


def kernel(outputs, labels, sessions):
    cand = _cand_table()
    negidx, posidx = _stage1(labels, sessions, cand)
    idx_all = jnp.concatenate(
        [negidx.reshape(-1), posidx.reshape(-1)]).reshape(_N_CHUNKS, _CHUNK)
    table = outputs.reshape(_B * _NCLASS)
    scores = _stage2(table, idx_all)
    loss = _stage3(scores)
    return loss.reshape(())
